# baseline (speedup 1.0000x reference)
import functools

import jax
import jax.numpy as jnp
from jax import lax
from jax.experimental import pallas as pl
from jax.experimental.pallas import tpu as pltpu
from jax.experimental.pallas import tpu_sc as plsc

NC = 2
NS = 16
NW = NC * NS
CHUNK = 128


def _sc_gather(uh, ih, jh, i_idx, j_idx, UL2, IL2, UV2, VF, IB):
    B = uh.shape[0]
    F = VF.shape[1]
    bpw = B // NW
    nch = bpw // CHUNK

    mesh = plsc.VectorSubcoreMesh(core_axis_name="c", subcore_axis_name="s")

    out_type = (
        jax.ShapeDtypeStruct((B, F), jnp.float32),
        jax.ShapeDtypeStruct((B, F), jnp.float32),
        jax.ShapeDtypeStruct((B, F), jnp.float32),
        jax.ShapeDtypeStruct((B, F), jnp.float32),
        jax.ShapeDtypeStruct((B, F), jnp.float32),
        jax.ShapeDtypeStruct((B, F), jnp.float32),
        jax.ShapeDtypeStruct((B,), jnp.float32),
        jax.ShapeDtypeStruct((B,), jnp.float32),
    )

    @functools.partial(
        pl.kernel,
        out_type=out_type,
        mesh=mesh,
        scratch_types=[
            pltpu.VMEM((CHUNK,), jnp.int32),
            pltpu.VMEM((CHUNK,), jnp.int32),
            pltpu.VMEM((CHUNK,), jnp.int32),
            pltpu.VMEM((CHUNK,), jnp.int32),
            pltpu.VMEM((CHUNK,), jnp.int32),
            pltpu.VMEM((CHUNK, F), jnp.float32),
            pltpu.VMEM((CHUNK, F), jnp.float32),
            pltpu.VMEM((CHUNK, F), jnp.float32),
            pltpu.VMEM((CHUNK, F), jnp.float32),
            pltpu.VMEM((CHUNK, F), jnp.float32),
            pltpu.VMEM((CHUNK, F), jnp.float32),
            pltpu.VMEM((CHUNK,), jnp.float32),
            pltpu.VMEM((CHUNK,), jnp.float32),
            pltpu.SemaphoreType.DMA,
        ],
        compiler_params=pltpu.CompilerParams(use_tc_tiling_on_sc=False),
    )
    def k(uh_hbm, ih_hbm, jh_hbm, i_hbm, j_hbm, UL, IL, UV, VFt, IBt,
          o_ul, o_ii, o_ij, o_uv, o_vi, o_vj, o_bi, o_bj,
          uh_c, ih_c, jh_c, i_c, j_c, bul, bii, bij, buv, bvi, bvj,
          bbi, bbj, sem):
        cid = lax.axis_index("c")
        sid = lax.axis_index("s")
        wid = sid * NC + cid
        base = wid * bpw
        for c in range(nch):
            off = base + c * CHUNK
            sl = pl.ds(off, CHUNK)
            pltpu.sync_copy(uh_hbm.at[sl], uh_c)
            pltpu.sync_copy(ih_hbm.at[sl], ih_c)
            pltpu.sync_copy(jh_hbm.at[sl], jh_c)
            pltpu.sync_copy(i_hbm.at[sl], i_c)
            pltpu.sync_copy(j_hbm.at[sl], j_c)
            cps = [
                pltpu.async_copy(UL.at[uh_c], bul, sem),
                pltpu.async_copy(IL.at[ih_c], bii, sem),
                pltpu.async_copy(IL.at[jh_c], bij, sem),
                pltpu.async_copy(UV.at[uh_c], buv, sem),
                pltpu.async_copy(VFt.at[i_c], bvi, sem),
                pltpu.async_copy(VFt.at[j_c], bvj, sem),
                pltpu.async_copy(IBt.at[i_c], bbi, sem),
                pltpu.async_copy(IBt.at[j_c], bbj, sem),
            ]
            for cp in cps:
                cp.wait()
            pltpu.sync_copy(bul, o_ul.at[sl])
            pltpu.sync_copy(bii, o_ii.at[sl])
            pltpu.sync_copy(bij, o_ij.at[sl])
            pltpu.sync_copy(buv, o_uv.at[sl])
            pltpu.sync_copy(bvi, o_vi.at[sl])
            pltpu.sync_copy(bvj, o_vj.at[sl])
            pltpu.sync_copy(bbi, o_bi.at[sl])
            pltpu.sync_copy(bbj, o_bj.at[sl])

    return k(uh, ih, jh, i_idx, j_idx, UL2, IL2, UV2, VF, IB)


def _tc_compute(gul, gii, gij, guv, vfi, vfj, ibi, ibj, pu, pi, pj,
                W_proj, beta):
    B, F = gul.shape
    K = W_proj.shape[0]
    BLK = 1024
    NB = B // BLK
    r3 = lambda x: x.reshape(NB, 1, BLK)

    def body(gul_r, gii_r, gij_r, guv_r, vfi_r, vfj_r, ibi_r, ibj_r,
             pu_r, pi_r, pj_r, W_r, beta_r, o_r):
        def sel(g, p_r):
            lo = g[:, :K]
            hi = g[:, K:]
            m = p_r[0, 0, :].reshape(BLK, 1)
            return jnp.where(m != 0, hi, lo)

        ul = sel(gul_r[...], pu_r)
        dil = sel(gii_r[...], pi_r) - sel(gij_r[...], pj_r)
        uv = sel(guv_r[...], pu_r)
        dvf = vfi_r[...] - vfj_r[...]
        proj = lax.dot_general(dvf, W_r[...], (((1,), (1,)), ((), ())),
                               preferred_element_type=jnp.float32)
        lat = jnp.sum(ul * dil, axis=1)
        vis = jnp.sum(uv * proj, axis=1)
        bet = jnp.sum(dvf * beta_r[...], axis=1)
        o_r[0, 0, :] = ibi_r[0, 0, :] - ibj_r[0, 0, :] + lat + vis + bet

    bf = pl.BlockSpec((BLK, F), lambda b: (b, 0))
    bs = pl.BlockSpec((1, 1, BLK), lambda b: (b, 0, 0))
    out3 = pl.pallas_call(
        body,
        grid=(NB,),
        in_specs=[bf, bf, bf, bf, bf, bf, bs, bs, bs, bs, bs,
                  pl.BlockSpec((K, F), lambda b: (0, 0)),
                  pl.BlockSpec((1, F), lambda b: (0, 0))],
        out_specs=bs,
        out_shape=jax.ShapeDtypeStruct((NB, 1, BLK), jnp.float32),
    )(gul, gii, gij, guv, vfi, vfj, r3(ibi), r3(ibj),
      r3(pu), r3(pi), r3(pj), W_proj, beta)
    return out3.reshape(B)


def kernel(trg_batch, U_latent, I_latent, U_visual, W_proj, b_proj,
           beta_dash, user_bias, item_bias, visual_features):
    tb = trg_batch.astype(jnp.int32)
    u_idx = tb[:, 0]
    i_idx = tb[:, 1]
    j_idx = tb[:, 2]
    N = U_latent.shape[0]
    K = U_latent.shape[1]
    UL2 = U_latent.reshape(N // 2, 2 * K)
    IL2 = I_latent.reshape(N // 2, 2 * K)
    UV2 = U_visual.reshape(N // 2, 2 * K)
    gathered = _sc_gather(u_idx >> 1, i_idx >> 1, j_idx >> 1, i_idx, j_idx,
                          UL2, IL2, UV2, visual_features, item_bias)
    gul, gii, gij, guv, vfi, vfj, ibi, ibj = gathered
    return _tc_compute(gul, gii, gij, guv, vfi, vfj, ibi, ibj,
                       (u_idx & 1), (i_idx & 1), (j_idx & 1),
                       W_proj, beta_dash)
